# Initial kernel scaffold; baseline (speedup 1.0000x reference)
import jax
import jax.numpy as jnp
from jax.experimental import pallas as pl


def kernel(h, edge_index, snorm_n, W1, b1, W2, b2, gamma, beta):
    raise NotImplementedError("write your pallas kernel here")


# speedup vs baseline: 7.4931x; 7.4931x over previous
import functools

import jax
import jax.numpy as jnp
from jax import lax
from jax.experimental import pallas as pl
from jax.experimental.pallas import tpu as pltpu
from jax.experimental.pallas import tpu_sc as plsc

N = 10000
E = 320000
D = 128
BN_EPS = 1e-5

NC = 2
NS = 16
NW = NC * NS
EPW = E // NW
K = 80
NB = EPW // K
RPT = 640
N_PAD = NS * RPT

_mesh = plsc.VectorSubcoreMesh(core_axis_name="c", subcore_axis_name="s")


@functools.partial(
    pl.kernel,
    out_type=jax.ShapeDtypeStruct((NC, N_PAD, D), jnp.float32),
    mesh=_mesh,
    scratch_types=[
        pltpu.VMEM((NB, K), jnp.int32),
        pltpu.VMEM((NB, K), jnp.int32),
        pltpu.VMEM((K, D), jnp.float32),
        pltpu.VMEM_SHARED((N_PAD, D), jnp.float32),
        pltpu.SemaphoreType.DMA,
        pltpu.SemaphoreType.DMA,
    ],
)
def _seg_sum(h_hbm, src_hbm, dst_hbm, zeros_hbm, out_hbm,
             src_v, dst_v, rows_v, acc, gsem, ssem):
    cid = lax.axis_index("c")
    sid = lax.axis_index("s")
    wid = cid * NS + sid
    row0 = sid * RPT
    pltpu.sync_copy(zeros_hbm, acc.at[pl.ds(row0, RPT)])
    pltpu.sync_copy(src_hbm.at[wid], src_v)
    pltpu.sync_copy(dst_hbm.at[wid], dst_v)
    plsc.subcore_barrier()

    def body(b, carry):
        pltpu.async_copy(h_hbm.at[src_v.at[b]], rows_v, gsem).wait()
        pltpu.async_copy(rows_v, acc.at[dst_v.at[b]], ssem, add=True).wait()
        return carry

    lax.fori_loop(0, NB, body, 0)
    plsc.subcore_barrier()
    pltpu.sync_copy(acc.at[pl.ds(row0, RPT)],
                    out_hbm.at[cid, pl.ds(row0, RPT)])


def _dense(h_ref, p_ref, sn_ref, w1_ref, b1_ref, w2_ref, b2_ref,
           g_ref, bt_ref, o_ref):
    x = h_ref[...] + p_ref[0, :N] + p_ref[1, :N]
    a = lax.dot_general(x, w1_ref[...], (((1,), (0,)), ((), ())),
                        preferred_element_type=jnp.float32)
    a = jnp.maximum(a + b1_ref[...], 0.0)
    y = lax.dot_general(a, w2_ref[...], (((1,), (0,)), ((), ())),
                        preferred_element_type=jnp.float32)
    y = (y + b2_ref[...]) * sn_ref[...]
    mean = jnp.mean(y, axis=0, keepdims=True)
    var = jnp.mean((y - mean) ** 2, axis=0, keepdims=True)
    o = g_ref[...] * (y - mean) * lax.rsqrt(var + BN_EPS) + bt_ref[...]
    o_ref[...] = jnp.maximum(o, 0.0)


_dense_call = pl.pallas_call(
    _dense,
    out_shape=jax.ShapeDtypeStruct((N, D), jnp.float32),
)


def kernel(h, edge_index, snorm_n, W1, b1, W2, b2, gamma, beta):
    src = edge_index[0].reshape(NW, NB, K)
    dst = edge_index[1].reshape(NW, NB, K)
    zeros = jnp.zeros((RPT, D), jnp.float32)
    partials = _seg_sum(h, src, dst, zeros)
    return _dense_call(h, partials, snorm_n,
                       W1, b1.reshape(1, D), W2, b2.reshape(1, D),
                       gamma.reshape(1, D), beta.reshape(1, D))
